# baseline (device time: 416573 ns/iter reference)
import jax
import jax.numpy as jnp
from jax import lax
from jax.experimental import pallas as pl
from jax.experimental.pallas import tpu as pltpu

N_DEV = 4
M, K, N = 4096, 4096, 8192
KS = K // N_DEV
MH = M // 2
KH = KS // 2
NQ = 4
QN = N // NQ
BM, BN = 1024, 1024


def _body(
    scale_ref, x_ref, w_ref, o_ref,
    gx, gw, stage,
    loc_sems, sx_sems, rx_sems, sw_sems, rw_sems, out_sems,
):
    me = lax.axis_index("i")
    right = (me + 1) % N_DEV
    left = (me - 1) % N_DEV

    cp_x = pltpu.make_async_copy(
        x_ref, gx.at[:, pl.ds(me * KS, KS)], loc_sems.at[0]
    )
    cp_w = pltpu.make_async_copy(
        w_ref, gw.at[pl.ds(me * KS, KS)], loc_sems.at[1]
    )
    cp_x.start()
    cp_w.start()

    def x_slc(s, d):
        return gx.at[pl.ds(d * MH, MH), pl.ds(s * KS, KS)]

    def w_slc(q, s, d):
        return gw.at[pl.ds(s * KS + d * KH, KH), pl.ds(q * QN, QN)]

    def hop(h, slc, src0, ssem_at, rsem_at):
        sR = (me - h) % N_DEV
        sL = (me + h) % N_DEV
        sends, recvs = [], []
        for d, s, tgt in [(0, sR, right), (1, sL, left)]:
            src = src0(d) if h == 0 else slc(s, d)
            rdma = pltpu.make_async_remote_copy(
                src_ref=src,
                dst_ref=slc(s, d),
                send_sem=ssem_at(d, h),
                recv_sem=rsem_at(d, h),
                device_id=(tgt,),
                device_id_type=pl.DeviceIdType.MESH,
            )
            rdma.start()
            sends.append(rdma)
        for d, r in [(0, (me - h - 1) % N_DEV), (1, (me + h + 1) % N_DEV)]:
            recvs.append(
                pltpu.make_async_remote_copy(
                    src_ref=slc(r, d),
                    dst_ref=slc(r, d),
                    send_sem=ssem_at(d, h),
                    recv_sem=rsem_at(d, h),
                    device_id=(left,),
                    device_id_type=pl.DeviceIdType.MESH,
                )
            )
        return sends, recvs

    def x_hop(h):
        return hop(
            h, x_slc, lambda d: x_ref.at[pl.ds(d * MH, MH)],
            lambda d, hh: sx_sems.at[d, hh],
            lambda d, hh: rx_sems.at[d, hh],
        )

    def w_hop(q, h):
        return hop(
            h,
            lambda s, d: w_slc(q, s, d),
            lambda d: w_ref.at[pl.ds(d * KH, KH), pl.ds(q * QN, QN)],
            lambda d, hh: sw_sems.at[q, d, hh],
            lambda d, hh: rw_sems.at[q, d, hh],
        )

    scale = scale_ref[0]
    pending = []

    def tile(q, t):
        i = (q * 8 + t) % 2
        if len(pending) >= 2:
            pending.pop(0).wait()
        mb, nb = t % 4, t // 4
        col = q * QN + nb * BN
        y = jnp.dot(
            gx[pl.ds(mb * BM, BM), :],
            gw[:, pl.ds(col, BN)],
            preferred_element_type=jnp.float32,
        ) * scale
        stage[i] = y * jax.nn.sigmoid(y)
        cp = pltpu.make_async_copy(
            stage.at[i], o_ref.at[pl.ds(mb * BM, BM), pl.ds(col, BN)],
            out_sems.at[i],
        )
        cp.start()
        pending.append(cp)

    all_sends = []

    for h in range(N_DEV - 1):
        s, r = x_hop(h)
        all_sends += s
        for rc in r:
            rc.wait_recv()
    for h in range(N_DEV - 1):
        s, r = w_hop(0, h)
        all_sends += s
        for rc in r:
            rc.wait_recv()
    cp_x.wait()
    cp_w.wait()

    for q in range(1, NQ):
        plan = [(0, [0, 1]), (1, [2, 3]), (2, [4, 5, 6, 7])]
        for h, tiles in plan:
            s, r = w_hop(q, h)
            all_sends += s
            for t in tiles:
                tile(q - 1, t)
            for rc in r:
                rc.wait_recv()

    for t in range(8):
        tile(NQ - 1, t)

    for cp in pending:
        cp.wait()
    for rdma in all_sends:
        rdma.wait_send()


def kernel(x, w_mat, scale_x, scale_w):
    x8 = x.astype(jnp.float8_e4m3fn)
    w8 = w_mat.astype(jnp.float8_e5m2)
    scale = (scale_x * scale_w).astype(jnp.float32)

    return pl.pallas_call(
        _body,
        out_shape=jax.ShapeDtypeStruct((M, N), jnp.float32),
        in_specs=[
            pl.BlockSpec(memory_space=pltpu.SMEM),
            pl.BlockSpec(memory_space=pl.ANY),
            pl.BlockSpec(memory_space=pl.ANY),
        ],
        out_specs=pl.BlockSpec(memory_space=pl.ANY),
        scratch_shapes=[
            pltpu.VMEM((M, K), jnp.float8_e4m3fn),
            pltpu.VMEM((K, N), jnp.float8_e5m2),
            pltpu.VMEM((2, BM, BN), jnp.float32),
            pltpu.SemaphoreType.DMA((2,)),
            pltpu.SemaphoreType.DMA((2, N_DEV - 1)),
            pltpu.SemaphoreType.DMA((2, N_DEV - 1)),
            pltpu.SemaphoreType.DMA((NQ, 2, N_DEV - 1)),
            pltpu.SemaphoreType.DMA((NQ, 2, N_DEV - 1)),
            pltpu.SemaphoreType.DMA((2,)),
        ],
        compiler_params=pltpu.CompilerParams(
            vmem_limit_bytes=100 * 1024 * 1024,
        ),
    )(scale, x8, w8)
